# baseline (speedup 1.0000x reference)
import functools

import jax
import jax.numpy as jnp
from jax import lax
from jax.experimental import pallas as pl
from jax.experimental.pallas import tpu as pltpu
from jax.experimental.pallas import tpu_sc as plsc

NUM_CORES = 2
NUM_SUBCORES = 16
NUM_WORKERS = NUM_CORES * NUM_SUBCORES

VOCAB = 100000
EMBED = 32
BATCH = 16384
PROWS = VOCAB // 4
NUM_JOBS = VOCAB // 128
JOBS_PER_WORKER = 25
TAIL_COL = NUM_JOBS * 128

_mesh = plsc.VectorSubcoreMesh(core_axis_name="c", subcore_axis_name="s")
_params = pltpu.CompilerParams(use_tc_tiling_on_sc=True, needs_layout_passes=False)


@functools.partial(
    pl.kernel,
    mesh=_mesh,
    out_type=jax.ShapeDtypeStruct((PROWS, 128), jnp.float32),
    compiler_params=_params,
    scratch_types=[
        pltpu.VMEM((2, 32, 128), jnp.float32),
        pltpu.VMEM((2, 32, 128), jnp.float32),
        pltpu.SemaphoreType.DMA,
        pltpu.SemaphoreType.DMA,
        pltpu.SemaphoreType.DMA,
        pltpu.SemaphoreType.DMA,
    ],
)
def _repack(table_t_hbm, tail_hbm, packed_hbm, slab_v, slabt_v, in0, in1, out0, out1):
    wid = lax.axis_index("s") * NUM_CORES + lax.axis_index("c")
    iota = lax.iota(jnp.int32, 16)
    sem_in = (in0, in1)
    sem_out = (out0, out1)

    def job(i):
        return i * NUM_WORKERS + wid

    def col_start(i):
        return pl.multiple_of(job(i) * 128, 128)

    def active(i):
        return job(i) < NUM_JOBS

    def fire_in(i, par):
        @pl.when(active(i))
        def _():
            pltpu.async_copy(
                table_t_hbm.at[:, pl.ds(col_start(i), 128)],
                slab_v.at[par],
                sem_in[par],
            )

    def step(i, par):
        @pl.when(active(i))
        def _():
            pltpu.make_async_copy(
                table_t_hbm.at[:, pl.ds(col_start(i), 128)],
                slab_v.at[par],
                sem_in[par],
            ).wait()

            @pl.when(i >= 2)
            def _():
                pltpu.make_async_copy(
                    slabt_v.at[par], packed_hbm.at[pl.ds(0, 32)], sem_out[par]
                ).wait()

            src = slab_v.at[par]
            dst = slabt_v.at[par]
            for l in range(128):
                col = jnp.full((16,), l, jnp.int32)
                lo = plsc.load_gather(src, [iota, col])
                hi = plsc.load_gather(src, [iota + 16, col])
                dst[l // 4, pl.ds((l % 4) * 32, 16)] = lo
                dst[l // 4, pl.ds((l % 4) * 32 + 16, 16)] = hi

            fire_in(i + 2, par)
            pltpu.async_copy(
                dst,
                packed_hbm.at[pl.ds(pl.multiple_of(job(i) * 32, 32), 32)],
                sem_out[par],
            )

    fire_in(0, 0)
    fire_in(1, 1)

    def pair(p, _):
        step(2 * p, 0)
        step(2 * p + 1, 1)
        return 0

    lax.fori_loop(0, JOBS_PER_WORKER // 2, pair, 0)
    step(JOBS_PER_WORKER - 1, 0)

    pltpu.make_async_copy(
        slabt_v.at[0], packed_hbm.at[pl.ds(0, 32)], sem_out[0]
    ).wait()
    pltpu.make_async_copy(
        slabt_v.at[1], packed_hbm.at[pl.ds(0, 32)], sem_out[1]
    ).wait()

    @pl.when(wid == NUM_WORKERS - 1)
    def _tail():
        pltpu.sync_copy(tail_hbm, slabt_v.at[0, pl.ds(0, 8)])
        pltpu.sync_copy(
            slabt_v.at[0, pl.ds(0, 8)],
            packed_hbm.at[pl.ds(TAIL_COL // 4, 8)],
        )


@functools.partial(
    pl.kernel,
    mesh=_mesh,
    out_type=jax.ShapeDtypeStruct((2 * EMBED, BATCH), jnp.float32),
    compiler_params=_params,
    scratch_types=[
        pltpu.VMEM((8, 128), jnp.int32),
        pltpu.VMEM((8, 128), jnp.int32),
        pltpu.VMEM((2, 128, 128), jnp.float32),
        pltpu.VMEM((2, 64, 128), jnp.float32),
        pltpu.SemaphoreType.DMA,
        pltpu.SemaphoreType.DMA,
        pltpu.SemaphoreType.DMA,
        pltpu.SemaphoreType.DMA,
    ],
)
def _gather(idx2_hbm, packed_hbm, out_t_hbm, idx_v, dmaidx_v, g_v, slab_v, g0, g1, o0, o1):
    wid = lax.axis_index("s") * NUM_CORES + lax.axis_index("c")
    sem_g = (g0, g1)
    sem_o = (o0, o1)
    pltpu.sync_copy(idx2_hbm.at[pl.ds(pl.multiple_of(8 * wid, 8), 8)], idx_v)

    for r in range(8):
        def to_packed(k, _):
            e = idx_v[r, pl.ds(16 * k, 16)]
            dmaidx_v[r, pl.ds(16 * k, 16)] = lax.shift_right_logical(e, 2)
            return 0

        lax.fori_loop(0, 8, to_packed, 0, unroll=8)

    for r in range(2):
        pltpu.async_copy(packed_hbm.at[dmaidx_v.at[r]], g_v.at[r], sem_g[r])

    for r in range(8):
        a = r % 2
        c_local = r // 2
        par = r % 2
        if a == 0 and c_local >= 2:
            pltpu.make_async_copy(
                slab_v.at[c_local % 2],
                out_t_hbm.at[:, pl.ds(0, 128)],
                sem_o[c_local % 2],
            ).wait()
        pltpu.make_async_copy(
            packed_hbm.at[dmaidx_v.at[r]], g_v.at[par], sem_g[par]
        ).wait()

        def extract(k, _):
            l16 = lax.iota(jnp.int32, 16) + 16 * k
            e = idx_v[r, pl.ds(16 * k, 16)]
            lane0 = lax.shift_left(jnp.bitwise_and(e, 3), 5)
            for c_out in range(EMBED):
                vals = plsc.load_gather(g_v.at[par], [l16, lane0 + c_out])
                slab_v[c_local % 2, a * EMBED + c_out, pl.ds(16 * k, 16)] = vals
            return 0

        lax.fori_loop(0, 8, extract, 0)

        if r + 2 < 8:
            pltpu.async_copy(
                packed_hbm.at[dmaidx_v.at[r + 2]], g_v.at[par], sem_g[par]
            )
        if a == 1:
            pltpu.async_copy(
                slab_v.at[c_local % 2],
                out_t_hbm.at[
                    :, pl.ds(pl.multiple_of((4 * wid + c_local) * 128, 128), 128)
                ],
                sem_o[c_local % 2],
            )

    pltpu.make_async_copy(
        slab_v.at[0], out_t_hbm.at[:, pl.ds(0, 128)], sem_o[0]
    ).wait()
    pltpu.make_async_copy(
        slab_v.at[1], out_t_hbm.at[:, pl.ds(0, 128)], sem_o[1]
    ).wait()


def kernel(action_indices, embedding_table):
    idx2 = (
        action_indices.astype(jnp.int32)
        .reshape(128, 128, 2)
        .transpose(0, 2, 1)
        .reshape(256, 128)
    )
    tail_packed = embedding_table[TAIL_COL:].reshape(8, 128)
    packed = _repack(embedding_table.T, tail_packed)
    out_t = _gather(idx2, packed)
    return out_t.T


# speedup vs baseline: 1.3308x; 1.3308x over previous
import functools

import jax
import jax.numpy as jnp
from jax import lax
from jax.experimental import pallas as pl
from jax.experimental.pallas import tpu as pltpu
from jax.experimental.pallas import tpu_sc as plsc

NUM_CORES = 2
NUM_SUBCORES = 16
NUM_WORKERS = NUM_CORES * NUM_SUBCORES

VOCAB = 100000
EMBED = 32
BATCH = 16384
PROWS = VOCAB // 4
NUM_JOBS = VOCAB // 128
JOBS_PER_WORKER = 25
TAIL_COL = NUM_JOBS * 128

_mesh = plsc.VectorSubcoreMesh(core_axis_name="c", subcore_axis_name="s")
_params = pltpu.CompilerParams(use_tc_tiling_on_sc=True, needs_layout_passes=False)


@functools.partial(
    pl.kernel,
    mesh=_mesh,
    out_type=jax.ShapeDtypeStruct((PROWS, 128), jnp.float32),
    compiler_params=_params,
    scratch_types=[
        pltpu.VMEM((2, 32, 128), jnp.float32),
        pltpu.VMEM((2, 32, 128), jnp.float32),
        pltpu.VMEM((16, 16), jnp.float32),
        pltpu.SemaphoreType.DMA,
        pltpu.SemaphoreType.DMA,
        pltpu.SemaphoreType.DMA,
        pltpu.SemaphoreType.DMA,
    ],
)
def _repack(table_t_hbm, tail_hbm, packed_hbm, slab_v, slabt_v, scr_v, in0, in1, out0, out1):
    wid = lax.axis_index("s") * NUM_CORES + lax.axis_index("c")
    iota = lax.iota(jnp.int32, 16)
    sem_in = (in0, in1)
    sem_out = (out0, out1)

    def job(i):
        return i * NUM_WORKERS + wid

    def col_start(i):
        return pl.multiple_of(job(i) * 128, 128)

    def active(i):
        return job(i) < NUM_JOBS

    def fire_in(i, par):
        @pl.when(active(i))
        def _():
            pltpu.async_copy(
                table_t_hbm.at[:, pl.ds(col_start(i), 128)],
                slab_v.at[par],
                sem_in[par],
            )

    def step(i, par):
        @pl.when(active(i))
        def _():
            pltpu.make_async_copy(
                table_t_hbm.at[:, pl.ds(col_start(i), 128)],
                slab_v.at[par],
                sem_in[par],
            ).wait()

            @pl.when(i >= 2)
            def _():
                pltpu.make_async_copy(
                    slabt_v.at[par], packed_hbm.at[pl.ds(0, 32)], sem_out[par]
                ).wait()

            src = slab_v.at[par]
            dst = slabt_v.at[par]

            def block(bi, _):
                jj = jnp.bitwise_and(bi, 1) * 16
                lbt = lax.shift_right_logical(bi, 1)
                l0 = lbt * 16
                for d in range(16):
                    cols = jnp.bitwise_and(iota + d, 15) + l0
                    scr_v[d, :] = plsc.load_gather(src, [iota + jj, cols])
                for m in range(16):
                    rows = jnp.bitwise_and(m - iota, 15)
                    o = plsc.load_gather(scr_v, [rows, iota])
                    dst[4 * lbt + (m // 4), pl.ds((m % 4) * 32 + jj, 16)] = o
                return 0

            lax.fori_loop(0, 16, block, 0)

            fire_in(i + 2, par)
            pltpu.async_copy(
                dst,
                packed_hbm.at[pl.ds(pl.multiple_of(job(i) * 32, 32), 32)],
                sem_out[par],
            )

    fire_in(0, 0)
    fire_in(1, 1)

    def pair(p, _):
        step(2 * p, 0)
        step(2 * p + 1, 1)
        return 0

    lax.fori_loop(0, JOBS_PER_WORKER // 2, pair, 0)
    step(JOBS_PER_WORKER - 1, 0)

    pltpu.make_async_copy(
        slabt_v.at[0], packed_hbm.at[pl.ds(0, 32)], sem_out[0]
    ).wait()
    pltpu.make_async_copy(
        slabt_v.at[1], packed_hbm.at[pl.ds(0, 32)], sem_out[1]
    ).wait()

    @pl.when(wid == NUM_WORKERS - 1)
    def _tail():
        pltpu.sync_copy(tail_hbm, slabt_v.at[0, pl.ds(0, 8)])
        pltpu.sync_copy(
            slabt_v.at[0, pl.ds(0, 8)],
            packed_hbm.at[pl.ds(TAIL_COL // 4, 8)],
        )


@functools.partial(
    pl.kernel,
    mesh=_mesh,
    out_type=jax.ShapeDtypeStruct((2 * EMBED, BATCH), jnp.float32),
    compiler_params=_params,
    scratch_types=[
        pltpu.VMEM((8, 128), jnp.int32),
        pltpu.VMEM((8, 128), jnp.int32),
        pltpu.VMEM((2, 128, 128), jnp.float32),
        pltpu.VMEM((2, 64, 128), jnp.float32),
        pltpu.VMEM((32, 16), jnp.float32),
        pltpu.SemaphoreType.DMA,
        pltpu.SemaphoreType.DMA,
        pltpu.SemaphoreType.DMA,
        pltpu.SemaphoreType.DMA,
    ],
)
def _gather(idx2_hbm, packed_hbm, out_t_hbm, idx_v, dmaidx_v, g_v, slab_v, scr_v, g0, g1, o0, o1):
    wid = lax.axis_index("s") * NUM_CORES + lax.axis_index("c")
    sem_g = (g0, g1)
    sem_o = (o0, o1)
    pltpu.sync_copy(idx2_hbm.at[pl.ds(pl.multiple_of(8 * wid, 8), 8)], idx_v)

    for r in range(8):
        def to_packed(k, _):
            e = idx_v[r, pl.ds(16 * k, 16)]
            dmaidx_v[r, pl.ds(16 * k, 16)] = lax.shift_right_logical(e, 2)
            return 0

        lax.fori_loop(0, 8, to_packed, 0, unroll=8)

    for r in range(2):
        pltpu.async_copy(packed_hbm.at[dmaidx_v.at[r]], g_v.at[r], sem_g[r])

    for r in range(8):
        a = r % 2
        c_local = r // 2
        par = r % 2
        if a == 0 and c_local >= 2:
            pltpu.make_async_copy(
                slab_v.at[c_local % 2],
                out_t_hbm.at[:, pl.ds(0, 128)],
                sem_o[c_local % 2],
            ).wait()
        pltpu.make_async_copy(
            packed_hbm.at[dmaidx_v.at[r]], g_v.at[par], sem_g[par]
        ).wait()

        def extract(k, _):
            iota = lax.iota(jnp.int32, 16)
            l16 = iota + 16 * k
            e = idx_v[r, pl.ds(16 * k, 16)]
            lane0 = lax.shift_left(jnp.bitwise_and(e, 3), 5)
            for h in (0, 1):
                for d in range(16):
                    cols = lane0 + (jnp.bitwise_and(iota + d, 15) + 16 * h)
                    scr_v[16 * h + d, :] = plsc.load_gather(
                        g_v.at[par], [l16, cols]
                    )
            for c_out in range(EMBED):
                rows = jnp.bitwise_and((c_out & 15) - iota, 15) + 16 * (c_out >> 4)
                vals = plsc.load_gather(scr_v, [rows, iota])
                slab_v[c_local % 2, a * EMBED + c_out, pl.ds(16 * k, 16)] = vals
            return 0

        lax.fori_loop(0, 8, extract, 0)

        if r + 2 < 8:
            pltpu.async_copy(
                packed_hbm.at[dmaidx_v.at[r + 2]], g_v.at[par], sem_g[par]
            )
        if a == 1:
            pltpu.async_copy(
                slab_v.at[c_local % 2],
                out_t_hbm.at[
                    :, pl.ds(pl.multiple_of((4 * wid + c_local) * 128, 128), 128)
                ],
                sem_o[c_local % 2],
            )

    pltpu.make_async_copy(
        slab_v.at[0], out_t_hbm.at[:, pl.ds(0, 128)], sem_o[0]
    ).wait()
    pltpu.make_async_copy(
        slab_v.at[1], out_t_hbm.at[:, pl.ds(0, 128)], sem_o[1]
    ).wait()


def kernel(action_indices, embedding_table):
    idx2 = (
        action_indices.astype(jnp.int32)
        .reshape(128, 128, 2)
        .transpose(0, 2, 1)
        .reshape(256, 128)
    )
    tail_packed = embedding_table[TAIL_COL:].reshape(8, 128)
    packed = _repack(embedding_table.T, tail_packed)
    out_t = _gather(idx2, packed)
    return out_t.T
